# Initial kernel scaffold; baseline (speedup 1.0000x reference)
import jax
import jax.numpy as jnp
from jax.experimental import pallas as pl


def kernel(x, tables, cts_w, cts_b, W1, b1, W2, b2):
    raise NotImplementedError("write your pallas kernel here")


# speedup vs baseline: 2.0229x; 2.0229x over previous
import functools

import jax
import jax.numpy as jnp
from jax import lax
from jax.experimental import pallas as pl
from jax.experimental.pallas import tpu as pltpu
from jax.experimental.pallas import tpu_sc as plsc

B = 16384
N_CAT = 26
N_CTS = 4
VOCAB = 100000
DIM = 16
H = 256
NC, NS = 2, 16
NW = NC * NS
CHUNK = B // NW
CAT_W = N_CAT * DIM
BLK = 2048


def _sc_gather(idx_perm, *tables):
    mesh = plsc.VectorSubcoreMesh(core_axis_name="c", subcore_axis_name="s")

    @functools.partial(
        pl.kernel,
        out_type=jax.ShapeDtypeStruct((B, CAT_W), jnp.float32),
        mesh=mesh,
        scratch_types=[
            pltpu.VMEM((N_CAT, CHUNK), jnp.int32),
            pltpu.VMEM((CHUNK, DIM), jnp.float32),
            pltpu.VMEM((CHUNK, DIM), jnp.float32),
            pltpu.SemaphoreType.DMA,
            pltpu.SemaphoreType.DMA,
        ],
        compiler_params=pltpu.CompilerParams(use_tc_tiling_on_sc=False),
    )
    def k(idx_hbm, *rest):
        tables_hbm = rest[:N_CAT]
        out_hbm, idx_v, rows0, rows1, sem0, sem1 = rest[N_CAT:]
        rows = (rows0, rows1)
        sems = (sem0, sem1)
        wid = lax.axis_index("s") * NC + lax.axis_index("c")
        base = wid * CHUNK
        pltpu.sync_copy(idx_hbm.at[wid], idx_v)
        handles = [None, None]
        handles[0] = pltpu.async_copy(
            tables_hbm[0].at[idx_v.at[0]], rows[0], sems[0])
        for t in range(N_CAT):
            if t + 1 < N_CAT:
                s = (t + 1) % 2
                handles[s] = pltpu.async_copy(
                    tables_hbm[t + 1].at[idx_v.at[t + 1]], rows[s], sems[s])
            handles[t % 2].wait()
            pltpu.sync_copy(
                rows[t % 2],
                out_hbm.at[pl.ds(base, CHUNK), pl.ds(t * DIM, DIM)])

    return k(idx_perm, *tables)


def _tc_mlp(e, xc, we, bcf, W1, b1, W2, b2):
    def body(e_ref, xc_ref, we_ref, bcf_ref, w1a_ref, w1b_ref, b1_ref,
             w2_ref, b2_ref, o_ref):
        ev = e_ref[...]
        lane = lax.broadcasted_iota(jnp.int32, ev.shape, 1)
        x_cat = jnp.where(lane < DIM, ev, jnp.maximum(ev, 0.0))
        x_cts = jnp.maximum(
            jnp.dot(xc_ref[...], we_ref[...],
                    preferred_element_type=jnp.float32) + bcf_ref[...], 0.0)
        h = jnp.maximum(
            jnp.dot(x_cat, w1a_ref[...], preferred_element_type=jnp.float32)
            + jnp.dot(x_cts, w1b_ref[...], preferred_element_type=jnp.float32)
            + b1_ref[...], 0.0)
        y = jnp.dot(h, w2_ref[...],
                    preferred_element_type=jnp.float32) + b2_ref[...]
        o_ref[...] = jnp.exp(y)

    W1a = W1[:CAT_W]
    W1b = W1[CAT_W:]
    return pl.pallas_call(
        body,
        grid=(B // BLK,),
        in_specs=[
            pl.BlockSpec((BLK, CAT_W), lambda i: (i, 0)),
            pl.BlockSpec((BLK, N_CTS), lambda i: (i, 0)),
            pl.BlockSpec((N_CTS, N_CTS * DIM), lambda i: (0, 0)),
            pl.BlockSpec((1, N_CTS * DIM), lambda i: (0, 0)),
            pl.BlockSpec((CAT_W, H), lambda i: (0, 0)),
            pl.BlockSpec((N_CTS * DIM, H), lambda i: (0, 0)),
            pl.BlockSpec((1, H), lambda i: (0, 0)),
            pl.BlockSpec((H, 1), lambda i: (0, 0)),
            pl.BlockSpec((1, 1), lambda i: (0, 0)),
        ],
        out_specs=pl.BlockSpec((BLK, 1), lambda i: (i, 0)),
        out_shape=jax.ShapeDtypeStruct((B, 1), jnp.float32),
    )(e, xc, we, bcf, W1a, W1b, b1, W2, b2)


def kernel(x, tables, cts_w, cts_b, W1, b1, W2, b2):
    idx = x[:, :N_CAT].astype(jnp.int32)
    idx_perm = (idx.T.reshape(N_CAT, NW, CHUNK)
                .transpose(1, 0, 2))
    e = _sc_gather(idx_perm, *tables)
    xc = x[:, N_CAT:]
    wc = jnp.stack([w[:, 0] for w in cts_w], axis=0)
    we = wc[:, None, :] * jnp.eye(N_CTS, dtype=x.dtype)[:, :, None]
    we = we.reshape(N_CTS, N_CTS * DIM)
    bcf = jnp.concatenate(cts_b, axis=0).reshape(1, N_CTS * DIM)
    return _tc_mlp(e, xc, we, bcf, W1, b1.reshape(1, H), W2,
                   b2.reshape(1, 1))
